# baseline (speedup 1.0000x reference)
import functools

import jax
import jax.numpy as jnp
from jax import lax
from jax.experimental import pallas as pl
from jax.experimental.pallas import tpu as pltpu
from jax.experimental.pallas import tpu_sc as plsc

B = 64
D = H = W = 64
OC = 2
NC = 2
NS = 16
NW = NC * NS
DW = D // NW
PH = DW * H
L = 16
GP = W // L
BC = 2
PR = 16

BT = 38
BS = B - BT
NCHUNK = BS // BC

DB = 8
ND = D // DB


def _sc_body(x_hbm, m_hbm, w_hbm, out_hbm,
             m_v, w0_v, w1_v, xa_v, xb_v, part_v, sema, semb):
    cid = lax.axis_index("c")
    sid = lax.axis_index("s")
    wid = sid * NC + cid
    ds = wid * DW

    pltpu.sync_copy(m_hbm.at[0, 0, pl.ds(ds, DW), :, :], m_v)
    pltpu.sync_copy(w_hbm.at[0, 0, pl.ds(ds, DW), :, :], w0_v)
    pltpu.sync_copy(w_hbm.at[1, 0, pl.ds(ds, DW), :, :], w1_v)

    def fold(ph, _):
        p = ph // H
        h = lax.rem(ph, H)
        for g in range(GP):
            sl = pl.ds(g * L, L)
            mv = m_v[p, h, sl]
            w0_v[p, h, sl] = w0_v[p, h, sl] * mv
            w1_v[p, h, sl] = w1_v[p, h, sl] * mv
        return 0

    lax.fori_loop(0, PH, fold, 0, unroll=2)

    bufs = (xa_v, xb_v)
    sems = (sema, semb)

    def start(c):
        pltpu.async_copy(
            x_hbm.at[pl.ds(BT + c * BC, BC), 0, pl.ds(ds, DW), :, :],
            bufs[c % 2], sems[c % 2])

    start(0)
    for c in range(NCHUNK):
        x_v = bufs[c % 2]
        pltpu.make_async_copy(
            x_hbm.at[pl.ds(BT + c * BC, BC), 0, pl.ds(ds, DW), :, :],
            x_v, sems[c % 2]).wait()
        if c + 1 < NCHUNK:
            start(c + 1)

        def row(ph, accs):
            p = ph // H
            h = lax.rem(ph, H)
            accs = list(accs)
            for g in range(GP):
                sl = pl.ds(g * L, L)
                w0v = w0_v[p, h, sl]
                w1v = w1_v[p, h, sl]
                for b in range(BC):
                    xv = x_v[b, p, h, sl]
                    xg = jnp.where(jnp.abs(xv) > 1e-8, xv, 0.0)
                    accs[2 * b] = accs[2 * b] + xg * w0v
                    accs[2 * b + 1] = accs[2 * b + 1] + xg * w1v
            return tuple(accs)

        z = jnp.zeros((L,), jnp.float32)
        accs = lax.fori_loop(0, PH, row, (z,) * (2 * BC))
        for b in range(BC):
            for o in range(OC):
                s = (c * BC + b) * OC + o
                part_v[s // 8, pl.ds((s % 8) * L, L)] = accs[2 * b + o]

    pltpu.sync_copy(part_v, out_hbm.at[wid])


def _combine_body(p_ref, b_ref, out_ref):
    t = jnp.sum(p_ref[...], axis=0)
    t = t.reshape(B * OC, L)
    out_ref[...] = jnp.sum(t, axis=1)[None, :] + b_ref[...]


def _tc_body(x_ref, m_ref, w_ref, b_ref, out_ref):
    d = pl.program_id(0)
    xv = x_ref[...]
    xg = jnp.where(jnp.abs(xv) > 1e-8, xv, 0.0) * m_ref[...]
    s0 = jnp.sum(xg * w_ref[0:1], axis=(1, 2, 3, 4))
    s1 = jnp.sum(xg * w_ref[1:2], axis=(1, 2, 3, 4))
    part = jnp.concatenate([s0[:, None], s1[:, None]], axis=1)

    @pl.when(d == 0)
    def _init():
        out_ref[...] = part + b_ref[...]

    @pl.when(d != 0)
    def _acc():
        out_ref[...] += part


def kernel(x, mask, weight, bias):
    mesh = plsc.VectorSubcoreMesh(core_axis_name="c", subcore_axis_name="s")
    partial = pl.kernel(
        _sc_body,
        out_type=jax.ShapeDtypeStruct((NW, PR, 128), jnp.float32),
        mesh=mesh,
        scratch_types=[
            pltpu.VMEM((DW, H, W), jnp.float32),
            pltpu.VMEM((DW, H, W), jnp.float32),
            pltpu.VMEM((DW, H, W), jnp.float32),
            pltpu.VMEM((BC, DW, H, W), jnp.float32),
            pltpu.VMEM((BC, DW, H, W), jnp.float32),
            pltpu.VMEM((PR, 128), jnp.float32),
            pltpu.SemaphoreType.DMA,
            pltpu.SemaphoreType.DMA,
        ],
        compiler_params=pltpu.CompilerParams(use_tc_tiling_on_sc=True),
    )(x, mask, weight)

    b2 = bias.reshape(1, OC)
    tc_out = pl.pallas_call(
        _tc_body,
        grid=(ND,),
        in_specs=[
            pl.BlockSpec((BT, 1, DB, H, W), lambda d: (0, 0, d, 0, 0)),
            pl.BlockSpec((1, 1, DB, H, W), lambda d: (0, 0, d, 0, 0)),
            pl.BlockSpec((OC, 1, DB, H, W), lambda d: (0, 0, d, 0, 0)),
            pl.BlockSpec((1, OC), lambda d: (0, 0)),
        ],
        out_specs=pl.BlockSpec((BT, OC), lambda d: (0, 0)),
        out_shape=jax.ShapeDtypeStruct((BT, OC), jnp.float32),
        compiler_params=pltpu.CompilerParams(
            dimension_semantics=("arbitrary",)),
    )(x, mask, weight, b2)

    bias_t = jnp.tile(bias, B).reshape(1, B * OC)
    sc_out = pl.pallas_call(
        _combine_body,
        out_shape=jax.ShapeDtypeStruct((1, B * OC), jnp.float32),
    )(partial, bias_t)
    return jnp.concatenate([tc_out, sc_out.reshape(B, OC)[:BS]], axis=0)


# speedup vs baseline: 1.0226x; 1.0226x over previous
import functools

import jax
import jax.numpy as jnp
from jax import lax
from jax.experimental import pallas as pl
from jax.experimental.pallas import tpu as pltpu
from jax.experimental.pallas import tpu_sc as plsc

B = 64
D = H = W = 64
OC = 2
NC = 2
NS = 16
NW = NC * NS
DW = D // NW
PH = DW * H
L = 16
GP = W // L
BC = 2
PR = 16

BT = 38
BS = B - BT
NCHUNK = BS // BC

DB = 8
ND = D // DB


def _sc_body(x_hbm, m_hbm, w_hbm, out_hbm,
             m_v, w0_v, w1_v, xa_v, xb_v, part_v, sema, semb):
    cid = lax.axis_index("c")
    sid = lax.axis_index("s")
    wid = sid * NC + cid
    ds = wid * DW

    pltpu.sync_copy(m_hbm.at[0, 0, pl.ds(ds, DW), :, :], m_v)
    pltpu.sync_copy(w_hbm.at[0, 0, pl.ds(ds, DW), :, :], w0_v)
    pltpu.sync_copy(w_hbm.at[1, 0, pl.ds(ds, DW), :, :], w1_v)

    def fold(ph, _):
        p = ph // H
        h = lax.rem(ph, H)
        for g in range(GP):
            sl = pl.ds(g * L, L)
            mv = m_v[p, h, sl]
            w0_v[p, h, sl] = w0_v[p, h, sl] * mv
            w1_v[p, h, sl] = w1_v[p, h, sl] * mv
        return 0

    lax.fori_loop(0, PH, fold, 0, unroll=2)

    bufs = (xa_v, xb_v)
    sems = (sema, semb)

    def start(c):
        pltpu.async_copy(
            x_hbm.at[pl.ds(BT + c * BC, BC), 0, pl.ds(ds, DW), :, :],
            bufs[c % 2], sems[c % 2])

    start(0)
    for c in range(NCHUNK):
        x_v = bufs[c % 2]
        pltpu.make_async_copy(
            x_hbm.at[pl.ds(BT + c * BC, BC), 0, pl.ds(ds, DW), :, :],
            x_v, sems[c % 2]).wait()
        if c + 1 < NCHUNK:
            start(c + 1)

        def row(ph, accs):
            p = ph // H
            h = lax.rem(ph, H)
            accs = list(accs)
            for g in range(GP):
                sl = pl.ds(g * L, L)
                w0v = w0_v[p, h, sl]
                w1v = w1_v[p, h, sl]
                for b in range(BC):
                    xv = x_v[b, p, h, sl]
                    xg = jnp.where(jnp.abs(xv) > 1e-8, xv, 0.0)
                    accs[2 * b] = accs[2 * b] + xg * w0v
                    accs[2 * b + 1] = accs[2 * b + 1] + xg * w1v
            return tuple(accs)

        z = jnp.zeros((L,), jnp.float32)
        accs = lax.fori_loop(0, PH, row, (z,) * (2 * BC))
        for b in range(BC):
            for o in range(OC):
                s = (c * BC + b) * OC + o
                part_v[s // 8, pl.ds((s % 8) * L, L)] = accs[2 * b + o]

    pltpu.sync_copy(part_v, out_hbm.at[wid])


def _combine_body(p_ref, tc_ref, b_ref, out_ref):
    t = jnp.sum(p_ref[...], axis=0)
    t = t.reshape(B * OC, L)
    sc = jnp.sum(t, axis=1) + b_ref[0]
    out_ref[...] = jnp.concatenate(
        [tc_ref[0], sc[:BS * OC]])[None, :]


def _tc_body(x_ref, m_ref, w_ref, b_ref, out_ref):
    d = pl.program_id(0)
    xv = x_ref[...]
    xg = jnp.where(jnp.abs(xv) > 1e-8, xv, 0.0) * m_ref[...]
    s0 = jnp.sum(xg * w_ref[0:1], axis=(1, 2, 3, 4))
    s1 = jnp.sum(xg * w_ref[1:2], axis=(1, 2, 3, 4))
    part = jnp.concatenate([s0[:, None], s1[:, None]], axis=1)

    @pl.when(d == 0)
    def _init():
        out_ref[...] = part + b_ref[...]

    @pl.when(d != 0)
    def _acc():
        out_ref[...] += part


def kernel(x, mask, weight, bias):
    mesh = plsc.VectorSubcoreMesh(core_axis_name="c", subcore_axis_name="s")
    partial = pl.kernel(
        _sc_body,
        out_type=jax.ShapeDtypeStruct((NW, PR, 128), jnp.float32),
        mesh=mesh,
        scratch_types=[
            pltpu.VMEM((DW, H, W), jnp.float32),
            pltpu.VMEM((DW, H, W), jnp.float32),
            pltpu.VMEM((DW, H, W), jnp.float32),
            pltpu.VMEM((BC, DW, H, W), jnp.float32),
            pltpu.VMEM((BC, DW, H, W), jnp.float32),
            pltpu.VMEM((PR, 128), jnp.float32),
            pltpu.SemaphoreType.DMA,
            pltpu.SemaphoreType.DMA,
        ],
        compiler_params=pltpu.CompilerParams(use_tc_tiling_on_sc=True),
    )(x, mask, weight)

    b2 = bias.reshape(1, OC)
    tc_out = pl.pallas_call(
        _tc_body,
        grid=(ND,),
        in_specs=[
            pl.BlockSpec((BT, 1, DB, H, W), lambda d: (0, 0, d, 0, 0)),
            pl.BlockSpec((1, 1, DB, H, W), lambda d: (0, 0, d, 0, 0)),
            pl.BlockSpec((OC, 1, DB, H, W), lambda d: (0, 0, d, 0, 0)),
            pl.BlockSpec((1, OC), lambda d: (0, 0)),
        ],
        out_specs=pl.BlockSpec((BT, OC), lambda d: (0, 0)),
        out_shape=jax.ShapeDtypeStruct((BT, OC), jnp.float32),
        compiler_params=pltpu.CompilerParams(
            dimension_semantics=("arbitrary",)),
    )(x, mask, weight, b2)

    bias_t = jnp.tile(bias, B).reshape(1, B * OC)
    out = pl.pallas_call(
        _combine_body,
        out_shape=jax.ShapeDtypeStruct((1, B * OC), jnp.float32),
    )(partial, tc_out.reshape(1, BT * OC), bias_t)
    return out.reshape(B, OC)


# speedup vs baseline: 1.0356x; 1.0127x over previous
import jax
import jax.numpy as jnp
from jax import lax
from jax.experimental import pallas as pl
from jax.experimental.pallas import tpu as pltpu
from jax.experimental.pallas import tpu_sc as plsc

B = 64
D = H = W = 64
OC = 2
NC = 2
NS = 16
NW = NC * NS
DW = D // NW
PH = DW * H
L = 16
GP = W // L
BC = 2
PR = 16

BT = 38
BS = B - BT
NCHUNK = BS // BC

DB = 8
ND = D // DB


def _sc_body(x_hbm, m_hbm, w_hbm, out_hbm,
             m_v, w0_v, w1_v, xa_v, xb_v, part_v, sema, semb):
    cid = lax.axis_index("c")
    sid = lax.axis_index("s")
    wid = sid * NC + cid
    ds = wid * DW

    pltpu.sync_copy(m_hbm.at[0, 0, pl.ds(ds, DW), :, :], m_v)
    pltpu.sync_copy(w_hbm.at[0, 0, pl.ds(ds, DW), :, :], w0_v)
    pltpu.sync_copy(w_hbm.at[1, 0, pl.ds(ds, DW), :, :], w1_v)

    def fold(ph, _):
        p = ph // H
        h = lax.rem(ph, H)
        for g in range(GP):
            sl = pl.ds(g * L, L)
            mv = m_v[p, h, sl]
            w0_v[p, h, sl] = w0_v[p, h, sl] * mv
            w1_v[p, h, sl] = w1_v[p, h, sl] * mv
        return 0

    lax.fori_loop(0, PH, fold, 0, unroll=2)

    bufs = (xa_v, xb_v)
    sems = (sema, semb)

    def start(c):
        pltpu.async_copy(
            x_hbm.at[pl.ds(BT + c * BC, BC), 0, pl.ds(ds, DW), :, :],
            bufs[c % 2], sems[c % 2])

    start(0)
    for c in range(NCHUNK):
        x_v = bufs[c % 2]
        pltpu.make_async_copy(
            x_hbm.at[pl.ds(BT + c * BC, BC), 0, pl.ds(ds, DW), :, :],
            x_v, sems[c % 2]).wait()
        if c + 1 < NCHUNK:
            start(c + 1)

        def row(ph, accs):
            p = ph // H
            h = lax.rem(ph, H)
            accs = list(accs)
            for g in range(GP):
                sl = pl.ds(g * L, L)
                w0v = w0_v[p, h, sl]
                w1v = w1_v[p, h, sl]
                for b in range(BC):
                    xv = x_v[b, p, h, sl]
                    xg = jnp.where(jnp.abs(xv) > 1e-8, xv, 0.0)
                    accs[2 * b] = accs[2 * b] + xg * w0v
                    accs[2 * b + 1] = accs[2 * b + 1] + xg * w1v
            return tuple(accs)

        z = jnp.zeros((L,), jnp.float32)
        accs = lax.fori_loop(0, PH, row, (z,) * (2 * BC))
        for b in range(BC):
            for o in range(OC):
                s = (c * BC + b) * OC + o
                part_v[s // 8, pl.ds((s % 8) * L, L)] = accs[2 * b + o]

    pltpu.sync_copy(part_v, out_hbm.at[wid])


def _combine_body(p_ref, b_ref, out_ref):
    t = jnp.sum(p_ref[...], axis=0)
    t = t.reshape(B * OC, L)
    out_ref[...] = jnp.sum(t, axis=1)[None, :] + b_ref[...]


def _tc_body(x_ref, m_ref, w_ref, b_ref, out_ref):
    d = pl.program_id(0)
    xv = x_ref[...]
    xg = jnp.where(jnp.abs(xv) > 1e-8, xv, 0.0) * m_ref[...]
    s0 = jnp.sum(xg * w_ref[0:1], axis=(1, 2, 3, 4))
    s1 = jnp.sum(xg * w_ref[1:2], axis=(1, 2, 3, 4))
    part = jnp.concatenate([s0[:, None], s1[:, None]], axis=1)

    @pl.when(d == 0)
    def _init():
        out_ref[...] = part + b_ref[...]

    @pl.when(d != 0)
    def _acc():
        out_ref[...] += part


def kernel(x, mask, weight, bias):
    mesh = plsc.VectorSubcoreMesh(core_axis_name="c", subcore_axis_name="s")
    partial = pl.kernel(
        _sc_body,
        out_type=jax.ShapeDtypeStruct((NW, PR, 128), jnp.float32),
        mesh=mesh,
        scratch_types=[
            pltpu.VMEM((DW, H, W), jnp.float32),
            pltpu.VMEM((DW, H, W), jnp.float32),
            pltpu.VMEM((DW, H, W), jnp.float32),
            pltpu.VMEM((BC, DW, H, W), jnp.float32),
            pltpu.VMEM((BC, DW, H, W), jnp.float32),
            pltpu.VMEM((PR, 128), jnp.float32),
            pltpu.SemaphoreType.DMA,
            pltpu.SemaphoreType.DMA,
        ],
        compiler_params=pltpu.CompilerParams(use_tc_tiling_on_sc=True),
    )(x, mask, weight)

    b2 = bias.reshape(1, OC)
    tc_out = pl.pallas_call(
        _tc_body,
        grid=(ND,),
        in_specs=[
            pl.BlockSpec((BT, 1, DB, H, W), lambda d: (0, 0, d, 0, 0)),
            pl.BlockSpec((1, 1, DB, H, W), lambda d: (0, 0, d, 0, 0)),
            pl.BlockSpec((OC, 1, DB, H, W), lambda d: (0, 0, d, 0, 0)),
            pl.BlockSpec((1, OC), lambda d: (0, 0)),
        ],
        out_specs=pl.BlockSpec((BT, OC), lambda d: (0, 0)),
        out_shape=jax.ShapeDtypeStruct((BT, OC), jnp.float32),
        compiler_params=pltpu.CompilerParams(
            dimension_semantics=("arbitrary",)),
    )(x, mask, weight, b2)

    bias_t = jnp.tile(bias, B).reshape(1, B * OC)
    sc_out = pl.pallas_call(
        _combine_body,
        out_shape=jax.ShapeDtypeStruct((1, B * OC), jnp.float32),
    )(partial, bias_t)
    return jnp.concatenate([tc_out, sc_out.reshape(B, OC)[:BS]], axis=0)
